# baseline (speedup 1.0000x reference)
import functools

import jax
import jax.numpy as jnp
from jax import lax
from jax.experimental import pallas as pl
from jax.experimental.pallas import tpu as pltpu
from jax.experimental.pallas import tpu_sc as plsc

B, L, W_CHARS = 1024, 50, 16
CHAR_SIZE = 128
CHAR_DIM = 30
CD_PAD = 32
HIDDEN = 1024
N_TOK = B * L
N_LOOK = N_TOK * W_CHARS
K_PAD = W_CHARS * CD_PAD
EMB_FLAT = N_TOK * K_PAD

_NC, _NS = 2, 16
_NW = _NC * _NS
_TPW = N_TOK // _NW
_CHT = 80
_NCH = _TPW // _CHT
_CHF = _CHT * K_PAD

_sc_mesh = plsc.VectorSubcoreMesh(
    core_axis_name="c", subcore_axis_name="s", num_cores=_NC, num_subcores=_NS
)


@functools.partial(
    pl.kernel,
    out_type=jax.ShapeDtypeStruct((EMB_FLAT,), jnp.float32),
    mesh=_sc_mesh,
    scratch_types=[
        pltpu.VMEM((_TPW * W_CHARS,), jnp.int32),
        pltpu.VMEM((CHAR_SIZE * CD_PAD,), jnp.float32),
        pltpu.VMEM((2, _CHF), jnp.float32),
        pltpu.SemaphoreType.DMA,
    ],
    compiler_params=pltpu.CompilerParams(
        use_tc_tiling_on_sc=False, needs_layout_passes=False
    ),
)
def _sc_gather(idx_hbm, tab_hbm, emb_hbm, idx_v, tab_v, out_v, wsem):
    wid = lax.axis_index("s") * _NC + lax.axis_index("c")
    pltpu.sync_copy(tab_hbm, tab_v)
    pltpu.sync_copy(idx_hbm.at[wid], idx_v)
    base = wid * (_TPW * K_PAD)

    iota = lax.broadcasted_iota(jnp.int32, (16,), 0)
    iota16 = iota * W_CHARS
    iota512 = iota * K_PAD

    def write_desc(ch, b):
        return pltpu.make_async_copy(
            out_v.at[b],
            emb_hbm.at[pl.ds(base + ch * _CHF, _CHF)],
            wsem,
        )

    @pl.loop(0, _NCH, step=2)
    def _chunk(c0):
        for nb in range(2):
            ch = c0 + nb

            @pl.when(ch >= 2)
            def _():
                write_desc(ch - 2, nb).wait()

            @pl.loop(0, _CHT // 16)
            def _tg(tg):
                tok0 = ch * _CHT + tg * 16

                @pl.loop(0, W_CHARS)
                def _w(w):
                    chars = plsc.load_gather(idx_v, [iota16 + (tok0 * W_CHARS + w)])
                    taddr = chars * CD_PAD
                    s0 = tg * (16 * K_PAD) + w * CD_PAD
                    for c in range(CD_PAD):
                        v = plsc.load_gather(tab_v, [taddr + c])
                        plsc.store_scatter(out_v.at[nb], [iota512 + (s0 + c)], v)

            write_desc(ch, nb).start()

    for nb in range(2):
        write_desc(_NCH - 2 + nb, nb).wait()


_TB = 512


def _mm_body(e_ref, wt_ref, o_ref):
    e = e_ref[:].reshape(_TB, K_PAD)
    o_ref[:] = jnp.dot(
        e.astype(jnp.bfloat16), wt_ref[:], preferred_element_type=jnp.float32
    )


@jax.jit
def kernel(X, char_table, W_proj):
    idx = X.reshape(_NW, _TPW * W_CHARS)
    tab32 = jnp.pad(char_table, ((0, 0), (0, CD_PAD - CHAR_DIM))).reshape(-1)
    emb = _sc_gather(idx, tab32)

    wt = jnp.pad(
        W_proj.reshape(HIDDEN, W_CHARS, CHAR_DIM),
        ((0, 0), (0, 0), (0, CD_PAD - CHAR_DIM)),
    ).reshape(HIDDEN, K_PAD).T.astype(jnp.bfloat16)

    out = pl.pallas_call(
        _mm_body,
        grid=(N_TOK // _TB,),
        in_specs=[
            pl.BlockSpec((_TB * K_PAD // 128, 128), lambda i: (i, 0)),
            pl.BlockSpec((K_PAD, HIDDEN), lambda i: (0, 0)),
        ],
        out_specs=pl.BlockSpec((_TB, HIDDEN), lambda i: (i, 0)),
        out_shape=jax.ShapeDtypeStruct((N_TOK, HIDDEN), jnp.float32),
    )(emb.reshape(EMB_FLAT // 128, 128), wt)
    return out.reshape(B, L, HIDDEN)


# speedup vs baseline: 1.4478x; 1.4478x over previous
import functools

import jax
import jax.numpy as jnp
from jax import lax
from jax.experimental import pallas as pl
from jax.experimental.pallas import tpu as pltpu
from jax.experimental.pallas import tpu_sc as plsc

B, L, W_CHARS = 1024, 50, 16
CHAR_SIZE = 128
CHAR_DIM = 30
CD_PAD = 32
HIDDEN = 1024
N_TOK = B * L
N_LOOK = N_TOK * W_CHARS
K_PAD = W_CHARS * CD_PAD

_NC, _NS = 2, 16
_NW = _NC * _NS
_LPW = N_LOOK // _NW
_GLOOK = 1280
_NG = _LPW // _GLOOK

_sc_mesh = plsc.VectorSubcoreMesh(
    core_axis_name="c", subcore_axis_name="s", num_cores=_NC, num_subcores=_NS
)


@functools.partial(
    pl.kernel,
    out_type=jax.ShapeDtypeStruct((N_LOOK, CD_PAD), jnp.float32),
    mesh=_sc_mesh,
    scratch_types=[
        pltpu.VMEM((_LPW,), jnp.int32),
        pltpu.VMEM((2, _GLOOK, CD_PAD), jnp.float32),
        pltpu.SemaphoreType.DMA,
        pltpu.SemaphoreType.DMA,
    ],
    compiler_params=pltpu.CompilerParams(use_tc_tiling_on_sc=False),
)
def _sc_gather(idx_hbm, tab_hbm, emb_hbm, idx_v, rows_v, gsem, wsem):
    wid = lax.axis_index("s") * _NC + lax.axis_index("c")
    pltpu.sync_copy(idx_hbm.at[wid], idx_v)
    base = wid * _LPW

    def write_desc(g, b):
        return pltpu.make_async_copy(
            rows_v.at[b],
            emb_hbm.at[pl.ds(base + g * _GLOOK, _GLOOK)],
            wsem,
        )

    def gather_desc(g, b):
        return pltpu.make_async_copy(
            tab_hbm.at[idx_v.at[pl.ds(g * _GLOOK, _GLOOK)]],
            rows_v.at[b],
            gsem,
        )

    @pl.loop(0, _NG, step=2)
    def _group(g0):
        for nb in range(2):
            g = g0 + nb

            @pl.when(g >= 2)
            def _():
                write_desc(g - 2, nb).wait()

            gather_desc(g, nb).start()
            gather_desc(g, nb).wait()
            write_desc(g, nb).start()

    for nb in range(2):
        write_desc(_NG - 2 + nb, nb).wait()


_TB = 512


def _mm_body(e_ref, wt_ref, o_ref):
    o_ref[:] = jnp.dot(
        e_ref[:].astype(jnp.bfloat16), wt_ref[:], preferred_element_type=jnp.float32
    )


@jax.jit
def kernel(X, char_table, W_proj):
    idx = X.reshape(_NW, _LPW)
    tab32 = jnp.pad(char_table, ((0, 0), (0, CD_PAD - CHAR_DIM)))
    emb = _sc_gather(idx, tab32)

    wt = jnp.pad(
        W_proj.reshape(HIDDEN, W_CHARS, CHAR_DIM),
        ((0, 0), (0, 0), (0, CD_PAD - CHAR_DIM)),
    ).reshape(HIDDEN, K_PAD).T.astype(jnp.bfloat16)

    out = pl.pallas_call(
        _mm_body,
        grid=(N_TOK // _TB,),
        in_specs=[
            pl.BlockSpec((_TB, K_PAD), lambda i: (i, 0)),
            pl.BlockSpec((K_PAD, HIDDEN), lambda i: (0, 0)),
        ],
        out_specs=pl.BlockSpec((_TB, HIDDEN), lambda i: (i, 0)),
        out_shape=jax.ShapeDtypeStruct((N_TOK, HIDDEN), jnp.float32),
    )(emb.reshape(N_TOK, K_PAD), wt)
    return out.reshape(B, L, HIDDEN)


# speedup vs baseline: 1.6826x; 1.1622x over previous
import functools

import jax
import jax.numpy as jnp
from jax import lax
from jax.experimental import pallas as pl
from jax.experimental.pallas import tpu as pltpu
from jax.experimental.pallas import tpu_sc as plsc

B, L, W_CHARS = 1024, 50, 16
CHAR_SIZE = 128
CHAR_DIM = 30
CD_PAD = 32
HIDDEN = 1024
N_TOK = B * L
N_LOOK = N_TOK * W_CHARS
K_PAD = W_CHARS * CD_PAD

_NC, _NS = 2, 16
_NW = _NC * _NS
_LPW = N_LOOK // _NW
_GLOOK = 1280
_NG = _LPW // _GLOOK

_sc_mesh = plsc.VectorSubcoreMesh(
    core_axis_name="c", subcore_axis_name="s", num_cores=_NC, num_subcores=_NS
)


@functools.partial(
    pl.kernel,
    out_type=jax.ShapeDtypeStruct((N_LOOK, CD_PAD), jnp.bfloat16),
    mesh=_sc_mesh,
    scratch_types=[
        pltpu.VMEM((_LPW,), jnp.int32),
        pltpu.VMEM((2, _GLOOK, CD_PAD), jnp.bfloat16),
        pltpu.SemaphoreType.DMA,
        pltpu.SemaphoreType.DMA,
    ],
    compiler_params=pltpu.CompilerParams(use_tc_tiling_on_sc=False),
)
def _sc_gather(idx_hbm, tab_hbm, emb_hbm, idx_v, rows_v, gsem, wsem):
    wid = lax.axis_index("s") * _NC + lax.axis_index("c")
    pltpu.sync_copy(idx_hbm.at[wid], idx_v)
    base = wid * _LPW

    def write_desc(g, b):
        return pltpu.make_async_copy(
            rows_v.at[b],
            emb_hbm.at[pl.ds(base + g * _GLOOK, _GLOOK)],
            wsem,
        )

    def gather_desc(g, b):
        return pltpu.make_async_copy(
            tab_hbm.at[idx_v.at[pl.ds(g * _GLOOK, _GLOOK)]],
            rows_v.at[b],
            gsem,
        )

    @pl.loop(0, _NG, step=2)
    def _group(g0):
        for nb in range(2):
            g = g0 + nb

            @pl.when(g >= 2)
            def _():
                write_desc(g - 2, nb).wait()

            gather_desc(g, nb).start()
            gather_desc(g, nb).wait()
            write_desc(g, nb).start()

    for nb in range(2):
        write_desc(_NG - 2 + nb, nb).wait()


_BB = 8
_TB = _BB * L


def _mm_body(e_ref, wt_ref, o_ref):
    res = jnp.dot(e_ref[:], wt_ref[:], preferred_element_type=jnp.float32)
    o_ref[:] = res.reshape(_BB, L, HIDDEN)


@jax.jit
def kernel(X, char_table, W_proj):
    idx = X.reshape(_NW, _LPW)
    tab32 = jnp.pad(char_table, ((0, 0), (0, CD_PAD - CHAR_DIM))).astype(jnp.bfloat16)
    emb = _sc_gather(idx, tab32)

    wt = jnp.pad(
        W_proj.reshape(HIDDEN, W_CHARS, CHAR_DIM),
        ((0, 0), (0, 0), (0, CD_PAD - CHAR_DIM)),
    ).reshape(HIDDEN, K_PAD).T.astype(jnp.bfloat16)

    out = pl.pallas_call(
        _mm_body,
        grid=(N_TOK // _TB,),
        in_specs=[
            pl.BlockSpec((_TB, K_PAD), lambda i: (i, 0)),
            pl.BlockSpec((K_PAD, HIDDEN), lambda i: (0, 0)),
        ],
        out_specs=pl.BlockSpec((_BB, L, HIDDEN), lambda i: (i, 0, 0)),
        out_shape=jax.ShapeDtypeStruct((B, L, HIDDEN), jnp.float32),
    )(emb.reshape(N_TOK, K_PAD), wt)
    return out


# speedup vs baseline: 2.2803x; 1.3552x over previous
import functools

import jax
import jax.numpy as jnp
from jax import lax
from jax.experimental import pallas as pl
from jax.experimental.pallas import tpu as pltpu
from jax.experimental.pallas import tpu_sc as plsc

B, L, W_CHARS = 1024, 50, 16
CHAR_SIZE = 128
CHAR_DIM = 30
CD_PAD = 32
HIDDEN = 1024
N_TOK = B * L
N_LOOK = N_TOK * W_CHARS
K_PAD = W_CHARS * CD_PAD

_NC, _NS = 2, 16
_NW = _NC * _NS
_LPW = N_LOOK // _NW
_GLOOK = 1280
_NG = _LPW // _GLOOK

_sc_mesh = plsc.VectorSubcoreMesh(
    core_axis_name="c", subcore_axis_name="s", num_cores=_NC, num_subcores=_NS
)


@functools.partial(
    pl.kernel,
    out_type=jax.ShapeDtypeStruct((N_LOOK, CD_PAD), jnp.bfloat16),
    mesh=_sc_mesh,
    scratch_types=[
        pltpu.VMEM((_LPW,), jnp.int32),
        pltpu.VMEM((2, _GLOOK, CD_PAD), jnp.bfloat16),
        pltpu.SemaphoreType.DMA,
        pltpu.SemaphoreType.DMA,
    ],
    compiler_params=pltpu.CompilerParams(use_tc_tiling_on_sc=False),
)
def _sc_gather(idx_hbm, tab_hbm, emb_hbm, idx_v, rows_v, gsem, wsem):
    wid = lax.axis_index("s") * _NC + lax.axis_index("c")
    pltpu.sync_copy(idx_hbm.at[wid], idx_v)
    base = wid * _LPW

    def write_desc(g, b):
        return pltpu.make_async_copy(
            rows_v.at[b],
            emb_hbm.at[pl.ds(base + g * _GLOOK, _GLOOK)],
            wsem,
        )

    def gather_desc(g, b):
        return pltpu.make_async_copy(
            tab_hbm.at[idx_v.at[pl.ds(g * _GLOOK, _GLOOK)]],
            rows_v.at[b],
            gsem,
        )

    @pl.loop(0, _NG, step=2)
    def _group(g0):
        for nb in range(2):
            g = g0 + nb

            @pl.when(g >= 2)
            def _():
                write_desc(g - 2, nb).wait()

            gather_desc(g, nb).start()
            gather_desc(g, nb).wait()
            write_desc(g, nb).start()

    for nb in range(2):
        write_desc(_NG - 2 + nb, nb).wait()


_TB = B


def _mm_body(e_ref, wt_ref, o_ref):
    o_ref[0] = jnp.dot(e_ref[:], wt_ref[:], preferred_element_type=jnp.float32)


@jax.jit
def kernel(X, char_table, W_proj):
    idx = X.transpose(1, 0, 2).reshape(_NW, _LPW)
    tab32 = jnp.pad(char_table, ((0, 0), (0, CD_PAD - CHAR_DIM))).astype(jnp.bfloat16)
    emb = _sc_gather(idx, tab32)

    wt = jnp.pad(
        W_proj.reshape(HIDDEN, W_CHARS, CHAR_DIM),
        ((0, 0), (0, 0), (0, CD_PAD - CHAR_DIM)),
    ).reshape(HIDDEN, K_PAD).T.astype(jnp.bfloat16)

    out = pl.pallas_call(
        _mm_body,
        grid=(N_TOK // _TB,),
        in_specs=[
            pl.BlockSpec((_TB, K_PAD), lambda i: (i, 0)),
            pl.BlockSpec((K_PAD, HIDDEN), lambda i: (0, 0)),
        ],
        out_specs=pl.BlockSpec((1, B, HIDDEN), lambda i: (i, 0, 0)),
        out_shape=jax.ShapeDtypeStruct((L, B, HIDDEN), jnp.float32),
    )(emb.reshape(N_TOK, K_PAD), wt)
    return out.transpose(1, 0, 2)


# speedup vs baseline: 4.0526x; 1.7772x over previous
import functools

import jax
import jax.numpy as jnp
from jax import lax
from jax.experimental import pallas as pl
from jax.experimental.pallas import tpu as pltpu
from jax.experimental.pallas import tpu_sc as plsc

B, L, W_CHARS = 1024, 50, 16
CHAR_SIZE = 128
CHAR_DIM = 30
CD_PAD = 32
HIDDEN = 1024
N_TOK = B * L
N_PAIR = W_CHARS // 2
N_LOOK = N_TOK * N_PAIR
PCD = 2 * CD_PAD
K_PAD = W_CHARS * CD_PAD

_NC, _NS = 2, 16
_NW = _NC * _NS
_LPW = N_LOOK // _NW
_GLOOK = 1280
_NG = _LPW // _GLOOK

_sc_mesh = plsc.VectorSubcoreMesh(
    core_axis_name="c", subcore_axis_name="s", num_cores=_NC, num_subcores=_NS
)


@functools.partial(
    pl.kernel,
    out_type=jax.ShapeDtypeStruct((N_LOOK, PCD), jnp.bfloat16),
    mesh=_sc_mesh,
    scratch_types=[
        pltpu.VMEM((_LPW,), jnp.int32),
        pltpu.VMEM((2, _GLOOK, PCD), jnp.bfloat16),
        pltpu.SemaphoreType.DMA,
        pltpu.SemaphoreType.DMA,
    ],
    compiler_params=pltpu.CompilerParams(use_tc_tiling_on_sc=False),
)
def _sc_gather(idx_hbm, tab_hbm, emb_hbm, idx_v, rows_v, gsem, wsem):
    wid = lax.axis_index("s") * _NC + lax.axis_index("c")
    pltpu.sync_copy(idx_hbm.at[wid], idx_v)
    base = wid * _LPW

    def write_desc(g, b):
        return pltpu.make_async_copy(
            rows_v.at[b],
            emb_hbm.at[pl.ds(base + g * _GLOOK, _GLOOK)],
            wsem,
        )

    def gather_desc(g, b):
        return pltpu.make_async_copy(
            tab_hbm.at[idx_v.at[pl.ds(g * _GLOOK, _GLOOK)]],
            rows_v.at[b],
            gsem,
        )

    @pl.loop(0, _NG, step=2)
    def _group(g0):
        for nb in range(2):
            g = g0 + nb

            @pl.when(g >= 2)
            def _():
                write_desc(g - 2, nb).wait()

            gather_desc(g, nb).start()
            gather_desc(g, nb).wait()
            write_desc(g, nb).start()

    for nb in range(2):
        write_desc(_NG - 2 + nb, nb).wait()


_TB = B


def _mm_body(e_ref, wt_ref, o_ref):
    e = e_ref[:].reshape(_TB, K_PAD)
    o_ref[0] = jnp.dot(e, wt_ref[:], preferred_element_type=jnp.float32)


@jax.jit
def kernel(X, char_table, W_proj):
    xt = X.transpose(1, 0, 2)
    idx = (xt[..., 0::2] * CHAR_SIZE + xt[..., 1::2]).reshape(_NW, _LPW)
    tab32 = jnp.pad(char_table, ((0, 0), (0, CD_PAD - CHAR_DIM))).astype(jnp.bfloat16)
    tabp = jnp.concatenate(
        [jnp.repeat(tab32, CHAR_SIZE, axis=0), jnp.tile(tab32, (CHAR_SIZE, 1))],
        axis=1,
    )
    emb = _sc_gather(idx, tabp)

    wt = jnp.pad(
        W_proj.reshape(HIDDEN, W_CHARS, CHAR_DIM),
        ((0, 0), (0, 0), (0, CD_PAD - CHAR_DIM)),
    ).reshape(HIDDEN, K_PAD).T.astype(jnp.bfloat16)

    out = pl.pallas_call(
        _mm_body,
        grid=(N_TOK // _TB,),
        in_specs=[
            pl.BlockSpec((_TB * 2, K_PAD // 2), lambda i: (i, 0)),
            pl.BlockSpec((K_PAD, HIDDEN), lambda i: (0, 0)),
        ],
        out_specs=pl.BlockSpec((1, B, HIDDEN), lambda i: (i, 0, 0)),
        out_shape=jax.ShapeDtypeStruct((L, B, HIDDEN), jnp.float32),
    )(emb.reshape(N_TOK * 2, K_PAD // 2), wt)
    return out.transpose(1, 0, 2)
